# Initial kernel scaffold; baseline (speedup 1.0000x reference)
import jax
import jax.numpy as jnp
from jax.experimental import pallas as pl


def kernel(cls_0, cls_1, cls_2, box_0, box_1, box_2, gt_boxes, gt_labels, gt_batch_index):
    raise NotImplementedError("write your pallas kernel here")


# speedup vs baseline: 10.3721x; 10.3721x over previous
import functools

import jax
import jax.numpy as jnp
from jax.experimental import pallas as pl

_IMG = 640.0
_LVLS = ((80, 80, 8), (40, 40, 16), (20, 20, 32))
_NC = 80
_B = 8
_M = 64
_K = 10
_RAD = 2.5

_pcall = pl.pallas_call


def _anchor_xy(HW, W, s):
    a = jax.lax.broadcasted_iota(jnp.int32, (1, HW), 1)
    af = a.astype(jnp.float32)
    rowf = jnp.floor(af * (1.0 / W))
    colf = af - rowf * W
    cx = (colf + 0.5) * s
    cy = (rowf + 0.5) * s
    return a, cx, cy


def _make_assign(H, W, s):
    HW = H * W
    r = _RAD * s

    def body(gt_ref, gb_ref, out_ref):
        gt = gt_ref[...]
        x1 = gt[:, 0:1]
        y1 = gt[:, 1:2]
        x2 = gt[:, 2:3]
        y2 = gt[:, 3:4]
        gx1 = jnp.clip(x1 - r, 0.0, _IMG)
        gy1 = jnp.clip(y1 - r, 0.0, _IMG)
        gx2 = jnp.clip(x2 + r, 0.0, _IMG)
        gy2 = jnp.clip(y2 + r, 0.0, _IMG)
        gcx = (x1 + x2) / 2.0
        gcy = (y1 + y2) / 2.0
        a, cx, cy = _anchor_xy(HW, W, s)
        inside = (cx >= gx1) & (cx <= gx2) & (cy >= gy1) & (cy <= gy2)
        dist = jnp.abs(cx - gcx) + jnp.abs(cy - gcy)
        dm = jnp.where(inside, dist, 1e9)
        has = jnp.any(inside, axis=1, keepdims=True)
        gb = gb_ref[...]
        img = jax.lax.broadcasted_iota(jnp.int32, (1, _B), 1)
        eq_ib = gb == img
        any_in = jnp.any(eq_ib & has, axis=0, keepdims=True)
        gate = jnp.any(eq_ib & any_in, axis=1, keepdims=True)
        hit = jnp.zeros(dm.shape, dtype=jnp.bool_)
        for _ in range(_K):
            v = jnp.min(dm, axis=1, keepdims=True)
            idx = jnp.min(jnp.where(dm == v, a, HW), axis=1, keepdims=True)
            pick = a == idx
            hit = hit | pick
            dm = jnp.where(pick, 2e9, dm)
        mi = jax.lax.broadcasted_iota(jnp.int32, (_M, 1), 0)
        for i in range(_B):
            sel = hit & (gb == i) & gate
            out_ref[i:i + 1, :] = jnp.max(jnp.where(sel, mi, -1), axis=0,
                                          keepdims=True)

    return body


def _loss_body(cls0, cls1, cls2, bx0, bx1, bx2, m0, m1, m2, gt_ref, lab_ref,
               o_ref):
    sp = jnp.float32(0.0)
    sel = jnp.float32(0.0)
    bl = jnp.float32(0.0)
    npf = jnp.float32(0.0)
    for cref, bref, mref, (H, W, s) in ((cls0, bx0, m0, _LVLS[0]),
                                        (cls1, bx1, m1, _LVLS[1]),
                                        (cls2, bx2, m2, _LVLS[2])):
        HW = H * W
        cls = cref[0]
        bo = bref[0]
        mrow = mref[0]
        a, cx, cy = _anchor_xy(HW, W, s)
        pos = mrow >= 0
        sp = sp + jnp.sum(jnp.maximum(cls, 0.0) +
                          jnp.log1p(jnp.exp(-jnp.abs(cls))))
        g1 = jnp.zeros((1, HW), jnp.float32)
        g2 = g1
        g3 = g1
        g4 = g1
        lb = g1
        for g in range(_M):
            c = mrow == g
            g1 = jnp.where(c, gt_ref[g:g + 1, 0:1], g1)
            g2 = jnp.where(c, gt_ref[g:g + 1, 1:2], g2)
            g3 = jnp.where(c, gt_ref[g:g + 1, 2:3], g3)
            g4 = jnp.where(c, gt_ref[g:g + 1, 3:4], g4)
            lb = jnp.where(c, lab_ref[g:g + 1, 0:1], lb)
        ci = jax.lax.broadcasted_iota(jnp.int32, (_NC, 1), 0)
        tmask = (ci == lb.astype(jnp.int32)) & pos
        sel = sel + jnp.sum(jnp.where(tmask, cls, 0.0))
        px1 = jnp.clip(cx - bo[0:1], 0.0, _IMG)
        py1 = jnp.clip(cy - bo[1:2], 0.0, _IMG)
        px2 = jnp.clip(cx + bo[2:3], 0.0, _IMG)
        py2 = jnp.clip(cy + bo[3:4], 0.0, _IMG)
        tl = jnp.maximum(cx - g1, 0.0)
        tt = jnp.maximum(cy - g2, 0.0)
        tr = jnp.maximum(g3 - cx, 0.0)
        tb = jnp.maximum(g4 - cy, 0.0)
        tx1 = cx - tl
        ty1 = cy - tt
        tx2 = cx + tr
        ty2 = cy + tb
        ix1 = jnp.maximum(px1, tx1)
        iy1 = jnp.maximum(py1, ty1)
        ix2 = jnp.minimum(px2, tx2)
        iy2 = jnp.minimum(py2, ty2)
        inter = jnp.maximum(ix2 - ix1, 0.0) * jnp.maximum(iy2 - iy1, 0.0)
        pa = jnp.maximum(px2 - px1, 0.0) * jnp.maximum(py2 - py1, 0.0)
        ta = jnp.maximum(tx2 - tx1, 0.0) * jnp.maximum(ty2 - ty1, 0.0)
        iou = inter / (pa + ta - inter + 1e-06)
        posf = pos.astype(jnp.float32)
        bl = bl + jnp.sum((1.0 - iou) * posf)
        npf = npf + jnp.sum(posf)
    lane = jax.lax.broadcasted_iota(jnp.int32, (1, 128), 1)
    contrib = (jnp.where(lane == 0, sp, 0.0) +
               jnp.where(lane == 1, sel, 0.0) +
               jnp.where(lane == 2, bl, 0.0) +
               jnp.where(lane == 3, npf, 0.0))
    i = pl.program_id(0)

    @pl.when(i == 0)
    def _():
        o_ref[...] = contrib

    @pl.when(i > 0)
    def _():
        o_ref[...] = o_ref[...] + contrib


def kernel(cls_0, cls_1, cls_2, box_0, box_1, box_2, gt_boxes, gt_labels,
           gt_batch_index):
    gt = gt_boxes.astype(jnp.float32)
    gb = gt_batch_index.astype(jnp.int32).reshape(_M, 1)
    lab = gt_labels.astype(jnp.float32).reshape(_M, 1)

    m_levels = []
    for (H, W, s) in _LVLS:
        body = _make_assign(H, W, s)
        m = _pcall(
            body,
            out_shape=jax.ShapeDtypeStruct((_B, H * W), jnp.int32),
        )(gt, gb)
        m_levels.append(m.reshape(_B, 1, H * W))

    csh = [c.reshape(_B, _NC, H * W)
           for c, (H, W, s) in zip((cls_0, cls_1, cls_2), _LVLS)]
    bsh = [b.reshape(_B, 4, H * W)
           for b, (H, W, s) in zip((box_0, box_1, box_2), _LVLS)]

    in_specs = (
        [pl.BlockSpec((1, _NC, H * W), lambda i: (i, 0, 0))
         for (H, W, s) in _LVLS] +
        [pl.BlockSpec((1, 4, H * W), lambda i: (i, 0, 0))
         for (H, W, s) in _LVLS] +
        [pl.BlockSpec((1, 1, H * W), lambda i: (i, 0, 0))
         for (H, W, s) in _LVLS] +
        [pl.BlockSpec((_M, 4), lambda i: (0, 0)),
         pl.BlockSpec((_M, 1), lambda i: (0, 0))]
    )
    acc = _pcall(
        _loss_body,
        grid=(_B,),
        in_specs=in_specs,
        out_specs=pl.BlockSpec((1, 128), lambda i: (0, 0)),
        out_shape=jax.ShapeDtypeStruct((1, 128), jnp.float32),
    )(csh[0], csh[1], csh[2], bsh[0], bsh[1], bsh[2],
      m_levels[0], m_levels[1], m_levels[2], gt, lab)

    sp = acc[0, 0]
    sel = acc[0, 1]
    bl = acc[0, 2]
    npos = acc[0, 3]
    return (sp - sel + 2.5 * bl) / jnp.maximum(npos, 1.0)


# speedup vs baseline: 12.5530x; 1.2103x over previous
import functools

import jax
import jax.numpy as jnp
from jax.experimental import pallas as pl

_IMG = 640.0
_LVLS = ((80, 80, 8), (40, 40, 16), (20, 20, 32))
_NC = 80
_B = 8
_M = 64
_K = 10
_RAD = 2.5

_pcall = pl.pallas_call


def _anchor_xy(HW, W, s):
    a = jax.lax.broadcasted_iota(jnp.int32, (1, HW), 1)
    af = a.astype(jnp.float32)
    rowf = jnp.floor(af * (1.0 / W))
    colf = af - rowf * W
    cx = (colf + 0.5) * s
    cy = (rowf + 0.5) * s
    return a, cx, cy


def _assign_body(gt_ref, gb_ref, out0_ref, out1_ref, out2_ref):
    outs = (out0_ref, out1_ref, out2_ref)
    for (H, W, s), out_ref in zip(_LVLS, outs):
        _assign_level(H, W, s, gt_ref, gb_ref, out_ref)


def _assign_level(H, W, s, gt_ref, gb_ref, out_ref):
    HW = H * W
    r = _RAD * s
    if True:
        gt = gt_ref[...]
        x1 = gt[:, 0:1]
        y1 = gt[:, 1:2]
        x2 = gt[:, 2:3]
        y2 = gt[:, 3:4]
        gx1 = jnp.clip(x1 - r, 0.0, _IMG)
        gy1 = jnp.clip(y1 - r, 0.0, _IMG)
        gx2 = jnp.clip(x2 + r, 0.0, _IMG)
        gy2 = jnp.clip(y2 + r, 0.0, _IMG)
        gcx = (x1 + x2) / 2.0
        gcy = (y1 + y2) / 2.0
        a, cx, cy = _anchor_xy(HW, W, s)
        inside = (cx >= gx1) & (cx <= gx2) & (cy >= gy1) & (cy <= gy2)
        dist = jnp.abs(cx - gcx) + jnp.abs(cy - gcy)
        dm = jnp.where(inside, dist, 1e9)
        has = jnp.any(inside, axis=1, keepdims=True)
        gb = gb_ref[...]
        img = jax.lax.broadcasted_iota(jnp.int32, (1, _B), 1)
        eq_ib = gb == img
        any_in = jnp.any(eq_ib & has, axis=0, keepdims=True)
        gate = jnp.any(eq_ib & any_in, axis=1, keepdims=True)
        hit = jnp.zeros(dm.shape, dtype=jnp.bool_)
        for _ in range(_K):
            v = jnp.min(dm, axis=1, keepdims=True)
            idx = jnp.min(jnp.where(dm == v, a, HW), axis=1, keepdims=True)
            pick = a == idx
            hit = hit | pick
            dm = jnp.where(pick, 2e9, dm)
        mi = jax.lax.broadcasted_iota(jnp.int32, (_M, 1), 0)
        for i in range(_B):
            sel = hit & (gb == i) & gate
            out_ref[i:i + 1, :] = jnp.max(jnp.where(sel, mi, -1), axis=0,
                                          keepdims=True)


def _loss_body(cls0, cls1, cls2, bx0, bx1, bx2, m0, m1, m2, tbl_ref, o_ref):
    sp = jnp.float32(0.0)
    sel = jnp.float32(0.0)
    bl = jnp.float32(0.0)
    npf = jnp.float32(0.0)
    tbl = tbl_ref[...]
    ci = jax.lax.broadcasted_iota(jnp.int32, (_NC, 1), 0)
    oh = (ci == tbl[4:5, :].astype(jnp.int32)).astype(jnp.float32)
    for cref, bref, mref, (H, W, s) in ((cls0, bx0, m0, _LVLS[0]),
                                        (cls1, bx1, m1, _LVLS[1]),
                                        (cls2, bx2, m2, _LVLS[2])):
        HW = H * W
        cls = cref[0]
        bo = bref[0]
        mrow = mref[0]
        a, cx, cy = _anchor_xy(HW, W, s)
        pos = mrow >= 0
        sp = sp + jnp.sum(jnp.maximum(cls, 0.0) +
                          jnp.log1p(jnp.exp(-jnp.abs(cls))))
        mi = jax.lax.broadcasted_iota(jnp.int32, (_M, 1), 0)
        p1h = (mi == mrow).astype(jnp.float32)
        flds = jax.lax.dot_general(tbl, p1h, (((1,), (0,)), ((), ())),
                                   preferred_element_type=jnp.float32)
        g1 = flds[0:1]
        g2 = flds[1:2]
        g3 = flds[2:3]
        g4 = flds[3:4]
        q = jax.lax.dot_general(cls, p1h, (((1,), (1,)), ((), ())),
                                preferred_element_type=jnp.float32)
        sel = sel + jnp.sum(q * oh)
        px1 = jnp.clip(cx - bo[0:1], 0.0, _IMG)
        py1 = jnp.clip(cy - bo[1:2], 0.0, _IMG)
        px2 = jnp.clip(cx + bo[2:3], 0.0, _IMG)
        py2 = jnp.clip(cy + bo[3:4], 0.0, _IMG)
        tl = jnp.maximum(cx - g1, 0.0)
        tt = jnp.maximum(cy - g2, 0.0)
        tr = jnp.maximum(g3 - cx, 0.0)
        tb = jnp.maximum(g4 - cy, 0.0)
        tx1 = cx - tl
        ty1 = cy - tt
        tx2 = cx + tr
        ty2 = cy + tb
        ix1 = jnp.maximum(px1, tx1)
        iy1 = jnp.maximum(py1, ty1)
        ix2 = jnp.minimum(px2, tx2)
        iy2 = jnp.minimum(py2, ty2)
        inter = jnp.maximum(ix2 - ix1, 0.0) * jnp.maximum(iy2 - iy1, 0.0)
        pa = jnp.maximum(px2 - px1, 0.0) * jnp.maximum(py2 - py1, 0.0)
        ta = jnp.maximum(tx2 - tx1, 0.0) * jnp.maximum(ty2 - ty1, 0.0)
        iou = inter / (pa + ta - inter + 1e-06)
        posf = pos.astype(jnp.float32)
        bl = bl + jnp.sum((1.0 - iou) * posf)
        npf = npf + jnp.sum(posf)
    lane = jax.lax.broadcasted_iota(jnp.int32, (1, 128), 1)
    contrib = (jnp.where(lane == 0, sp, 0.0) +
               jnp.where(lane == 1, sel, 0.0) +
               jnp.where(lane == 2, bl, 0.0) +
               jnp.where(lane == 3, npf, 0.0))
    i = pl.program_id(0)

    @pl.when(i == 0)
    def _():
        o_ref[...] = contrib

    @pl.when(i > 0)
    def _():
        o_ref[...] = o_ref[...] + contrib


def kernel(cls_0, cls_1, cls_2, box_0, box_1, box_2, gt_boxes, gt_labels,
           gt_batch_index):
    gt = gt_boxes.astype(jnp.float32)
    gb = gt_batch_index.astype(jnp.int32).reshape(_M, 1)
    lab = gt_labels.astype(jnp.float32).reshape(1, _M)
    tbl = jnp.concatenate(
        [gt.T, lab, jnp.zeros((3, _M), jnp.float32)], axis=0)

    m_raw = _pcall(
        _assign_body,
        out_shape=[jax.ShapeDtypeStruct((_B, H * W), jnp.int32)
                   for (H, W, s) in _LVLS],
    )(gt, gb)
    m_levels = [m.reshape(_B, 1, H * W)
                for m, (H, W, s) in zip(m_raw, _LVLS)]

    csh = [c.reshape(_B, _NC, H * W)
           for c, (H, W, s) in zip((cls_0, cls_1, cls_2), _LVLS)]
    bsh = [b.reshape(_B, 4, H * W)
           for b, (H, W, s) in zip((box_0, box_1, box_2), _LVLS)]

    in_specs = (
        [pl.BlockSpec((1, _NC, H * W), lambda i: (i, 0, 0))
         for (H, W, s) in _LVLS] +
        [pl.BlockSpec((1, 4, H * W), lambda i: (i, 0, 0))
         for (H, W, s) in _LVLS] +
        [pl.BlockSpec((1, 1, H * W), lambda i: (i, 0, 0))
         for (H, W, s) in _LVLS] +
        [pl.BlockSpec((8, _M), lambda i: (0, 0))]
    )
    acc = _pcall(
        _loss_body,
        grid=(_B,),
        in_specs=in_specs,
        out_specs=pl.BlockSpec((1, 128), lambda i: (0, 0)),
        out_shape=jax.ShapeDtypeStruct((1, 128), jnp.float32),
    )(csh[0], csh[1], csh[2], bsh[0], bsh[1], bsh[2],
      m_levels[0], m_levels[1], m_levels[2], tbl)

    sp = acc[0, 0]
    sel = acc[0, 1]
    bl = acc[0, 2]
    npos = acc[0, 3]
    return (sp - sel + 2.5 * bl) / jnp.maximum(npos, 1.0)
